# baseline (speedup 1.0000x reference)
import functools

import jax
import jax.numpy as jnp
from jax import lax
from jax.experimental import pallas as pl
from jax.experimental.pallas import tpu as pltpu
from jax.experimental.pallas import tpu_sc as plsc

_D = 256
_K = 8192
_BN = 3072
_BK = 512
_NCHUNK = _K // _BK
_LANES = 128
_WINDOW_ENDS = (2736, 5472, 8192)


def _argmin_body(z_ref, et_ref, z2_ref, idx_ref, e2_ref):
    i = pl.program_id(0)

    @pl.when(i == 0)
    def _():
        et = et_ref[...]
        e2_ref[...] = jnp.sum(et * et, axis=0)

    z = z_ref[...]
    z2 = z2_ref[...][:, None]
    a = (2.0 * z).astype(jnp.bfloat16)
    bn = z.shape[0]
    lane = lax.broadcasted_iota(jnp.int32, (bn, _LANES), 1)
    inf = jnp.float32(jnp.inf)

    accv = jnp.full((bn, _LANES), inf, jnp.float32)
    acci = jnp.zeros((bn, _LANES), jnp.int32)
    rv = jnp.full((bn,), inf, jnp.float32)
    ri = jnp.zeros((bn,), jnp.int32)

    def upd(dg, idxg):
        nonlocal accv, acci
        better = dg < accv
        accv = jnp.where(better, dg, accv)
        acci = jnp.where(better, idxg, acci)

    def finish_window():
        nonlocal accv, acci, rv, ri
        vw = jnp.min(accv, axis=1)
        iw = jnp.min(jnp.where(accv == vw[:, None], acci, _K), axis=1)
        better = vw < rv
        rv = jnp.where(better, vw, rv)
        ri = jnp.where(better, iw, ri)
        rv = rv.astype(jnp.bfloat16).astype(jnp.float32)
        accv = jnp.full((bn, _LANES), inf, jnp.float32)
        acci = jnp.zeros((bn, _LANES), jnp.int32)

    def combine(pair_a, pair_b):
        va, ia = pair_a
        vb, ib = pair_b
        better = vb < va
        return jnp.where(better, vb, va), jnp.where(better, ib, ia)

    for c in range(_NCHUNK):
        e_c = et_ref[:, pl.ds(c * _BK, _BK)]
        m = lax.dot_general(a, e_c, (((1,), (0,)), ((), ())),
                            preferred_element_type=jnp.float32)
        d = (z2 - m) + e2_ref[pl.ds(c * _BK, _BK)][None, :]
        ngroup = _BK // _LANES
        has_cut = any(c * _BK < b < (c + 1) * _BK for b in _WINDOW_ENDS[:-1])
        if not has_cut:
            pairs = [(d[:, g * _LANES:(g + 1) * _LANES], lane + c * _BK + g * _LANES)
                     for g in range(ngroup)]
            while len(pairs) > 1:
                pairs = [combine(pairs[k], pairs[k + 1])
                         for k in range(0, len(pairs), 2)]
            upd(*pairs[0])
        else:
            for g in range(ngroup):
                s = c * _BK + g * _LANES
                dg = d[:, g * _LANES:(g + 1) * _LANES]
                idxg = lane + s
                cuts = [b for b in _WINDOW_ENDS[:-1] if s < b < s + _LANES]
                if not cuts:
                    upd(dg, idxg)
                else:
                    cut = cuts[0] - s
                    upd(jnp.where(lane < cut, dg, inf), idxg)
                    finish_window()
                    upd(jnp.where(lane >= cut, dg, inf), idxg)
    finish_window()
    idx_ref[...] = ri


def _compute_indices(z_flat, emb_t, z2, bn=_BN):
    n, d = z_flat.shape
    k = emb_t.shape[1]
    return pl.pallas_call(
        _argmin_body,
        grid=(n // bn,),
        in_specs=[
            pl.BlockSpec((bn, d), lambda i: (i, 0)),
            pl.BlockSpec((d, k), lambda i: (0, 0)),
            pl.BlockSpec((bn,), lambda i: (i,)),
        ],
        out_specs=pl.BlockSpec((bn,), lambda i: (i,)),
        out_shape=jax.ShapeDtypeStruct((n,), jnp.int32),
        scratch_shapes=[
            pltpu.VMEM((k,), jnp.float32),
        ],
        compiler_params=pltpu.CompilerParams(
            dimension_semantics=("arbitrary",),
        ),
    )(z_flat, emb_t, z2)


def _gather_rows(emb, idx):
    info = plsc.get_sparse_core_info()
    nc, ns = info.num_cores, info.num_subcores
    nw = nc * ns
    b = idx.shape[0]
    b_per_w = b // nw
    chunk = max(c for c in range(8, 129, 8) if b_per_w % c == 0)
    nchunk = b_per_w // chunk
    mesh = plsc.VectorSubcoreMesh(core_axis_name="c", subcore_axis_name="s")

    @functools.partial(
        pl.kernel, mesh=mesh,
        out_type=jax.ShapeDtypeStruct((b, _D), jnp.float32),
        scratch_types=[
            pltpu.VMEM((chunk,), jnp.int32),
            pltpu.VMEM((chunk,), jnp.int32),
            pltpu.VMEM((chunk, _D), jnp.float32),
            pltpu.VMEM((chunk, _D), jnp.float32),
            pltpu.SemaphoreType.DMA,
            pltpu.SemaphoreType.DMA,
            pltpu.SemaphoreType.DMA,
            pltpu.SemaphoreType.DMA,
        ],
    )
    def gk(table_hbm, idx_hbm, out_hbm, i0, i1, r0, r1, g0, g1, o0, o1):
        wid = lax.axis_index("s") * nc + lax.axis_index("c")
        base = wid * b_per_w
        idx_bufs = (i0, i1)
        row_bufs = (r0, r1)
        g_sems = (g0, g1)
        o_sems = (o0, o1)
        gathers = {}
        writes = {}
        for c in range(nchunk):
            off = base + c * chunk
            s = c % 2
            if c >= 2:
                writes[c - 2].wait()
            pltpu.sync_copy(idx_hbm.at[pl.ds(off, chunk)], idx_bufs[s])
            gathers[c] = pltpu.async_copy(
                table_hbm.at[idx_bufs[s]], row_bufs[s], g_sems[s])
            if c >= 1:
                p = c - 1
                gathers[p].wait()
                writes[p] = pltpu.async_copy(
                    row_bufs[p % 2], out_hbm.at[pl.ds(base + p * chunk, chunk)],
                    o_sems[p % 2])
        last = nchunk - 1
        gathers[last].wait()
        writes[last] = pltpu.async_copy(
            row_bufs[last % 2], out_hbm.at[pl.ds(base + last * chunk, chunk)],
            o_sems[last % 2])
        for c in (last - 1, last):
            if c >= 0 and c not in (last - 2,):
                writes[c].wait()

    return gk(emb, idx)


def kernel(z, embeddings):
    z_flat = z.reshape(-1, _D)
    z2 = jnp.sum(z_flat ** 2, axis=1)
    emb_t = embeddings.T
    qs = []
    for s in range(0, z_flat.shape[0], _BN):
        idx_s = _compute_indices(z_flat[s:s + _BN], emb_t, z2[s:s + _BN])
        qs.append(_gather_rows(embeddings, idx_s))
    return jnp.concatenate(qs, axis=0).reshape(z.shape)


# speedup vs baseline: 1.1921x; 1.1921x over previous
import functools

import jax
import jax.numpy as jnp
from jax import lax
from jax.experimental import pallas as pl
from jax.experimental.pallas import tpu as pltpu
from jax.experimental.pallas import tpu_sc as plsc

_D = 256
_K = 8192
_BN = 3072
_BK = 512
_NCHUNK = _K // _BK
_LANES = 128
_WINDOW_ENDS = (2736, 5472, 8192)


def _argmin_body(z_ref, et_ref, z2_ref, idx_ref, e2_ref):
    i = pl.program_id(0)

    @pl.when(i == 0)
    def _():
        et = et_ref[...]
        e2_ref[...] = jnp.sum(et * et, axis=0)

    z = z_ref[...]
    z2 = z2_ref[...][:, None]
    a = (2.0 * z).astype(jnp.bfloat16)
    bn = z.shape[0]
    lane = lax.broadcasted_iota(jnp.int32, (bn, _LANES), 1)
    inf = jnp.float32(jnp.inf)

    accv = jnp.full((bn, _LANES), inf, jnp.float32)
    acci = jnp.zeros((bn, _LANES), jnp.int32)
    rv = jnp.full((bn,), inf, jnp.float32)
    ri = jnp.zeros((bn,), jnp.int32)

    def upd(dg, idxg):
        nonlocal accv, acci
        better = dg < accv
        accv = jnp.where(better, dg, accv)
        acci = jnp.where(better, idxg, acci)

    def finish_window():
        nonlocal accv, acci, rv, ri
        vw = jnp.min(accv, axis=1)
        iw = jnp.min(jnp.where(accv == vw[:, None], acci, _K), axis=1)
        better = vw < rv
        rv = jnp.where(better, vw, rv)
        ri = jnp.where(better, iw, ri)
        rv = rv.astype(jnp.bfloat16).astype(jnp.float32)
        accv = jnp.full((bn, _LANES), inf, jnp.float32)
        acci = jnp.zeros((bn, _LANES), jnp.int32)

    def combine(pair_a, pair_b):
        va, ia = pair_a
        vb, ib = pair_b
        better = vb < va
        return jnp.where(better, vb, va), jnp.where(better, ib, ia)

    for c in range(_NCHUNK):
        e_c = et_ref[:, pl.ds(c * _BK, _BK)]
        m = lax.dot_general(a, e_c, (((1,), (0,)), ((), ())),
                            preferred_element_type=jnp.float32)
        d = (z2 - m) + e2_ref[pl.ds(c * _BK, _BK)][None, :]
        ngroup = _BK // _LANES
        has_cut = any(c * _BK < b < (c + 1) * _BK for b in _WINDOW_ENDS[:-1])
        if not has_cut:
            pairs = [(d[:, g * _LANES:(g + 1) * _LANES], lane + c * _BK + g * _LANES)
                     for g in range(ngroup)]
            while len(pairs) > 1:
                pairs = [combine(pairs[k], pairs[k + 1])
                         for k in range(0, len(pairs), 2)]
            upd(*pairs[0])
        else:
            for g in range(ngroup):
                s = c * _BK + g * _LANES
                dg = d[:, g * _LANES:(g + 1) * _LANES]
                idxg = lane + s
                cuts = [b for b in _WINDOW_ENDS[:-1] if s < b < s + _LANES]
                if not cuts:
                    upd(dg, idxg)
                else:
                    cut = cuts[0] - s
                    upd(jnp.where(lane < cut, dg, inf), idxg)
                    finish_window()
                    upd(jnp.where(lane >= cut, dg, inf), idxg)
    finish_window()
    idx_ref[...] = ri


def _compute_indices(z_flat, emb_t, z2, bn=_BN):
    n, d = z_flat.shape
    k = emb_t.shape[1]
    return pl.pallas_call(
        _argmin_body,
        grid=(n // bn,),
        in_specs=[
            pl.BlockSpec((bn, d), lambda i: (i, 0)),
            pl.BlockSpec((d, k), lambda i: (0, 0)),
            pl.BlockSpec((bn,), lambda i: (i,)),
        ],
        out_specs=pl.BlockSpec((bn,), lambda i: (i,)),
        out_shape=jax.ShapeDtypeStruct((n,), jnp.int32),
        scratch_shapes=[
            pltpu.VMEM((k,), jnp.float32),
        ],
        compiler_params=pltpu.CompilerParams(
            dimension_semantics=("arbitrary",),
        ),
    )(z_flat, emb_t, z2)


def _gather_rows(emb, idx):
    info = plsc.get_sparse_core_info()
    nc, ns = info.num_cores, info.num_subcores
    nw = nc * ns
    b = idx.shape[0]
    b_per_w = b // nw
    chunk = max(c for c in range(8, 129, 8) if b_per_w % c == 0)
    nchunk = b_per_w // chunk
    mesh = plsc.VectorSubcoreMesh(core_axis_name="c", subcore_axis_name="s")

    @functools.partial(
        pl.kernel, mesh=mesh,
        out_type=jax.ShapeDtypeStruct((b, _D), jnp.float32),
        scratch_types=[
            pltpu.VMEM((chunk,), jnp.int32),
            pltpu.VMEM((chunk,), jnp.int32),
            pltpu.VMEM((chunk, _D), jnp.float32),
            pltpu.VMEM((chunk, _D), jnp.float32),
            pltpu.SemaphoreType.DMA,
            pltpu.SemaphoreType.DMA,
            pltpu.SemaphoreType.DMA,
            pltpu.SemaphoreType.DMA,
        ],
    )
    def gk(table_hbm, idx_hbm, out_hbm, i0, i1, r0, r1, g0, g1, o0, o1):
        wid = lax.axis_index("s") * nc + lax.axis_index("c")
        base = wid * b_per_w
        idx_bufs = (i0, i1)
        row_bufs = (r0, r1)
        g_sems = (g0, g1)
        o_sems = (o0, o1)
        gathers = {}
        writes = {}
        for c in range(nchunk):
            off = base + c * chunk
            s = c % 2
            if c >= 2:
                writes[c - 2].wait()
            pltpu.sync_copy(idx_hbm.at[pl.ds(off, chunk)], idx_bufs[s])
            gathers[c] = pltpu.async_copy(
                table_hbm.at[idx_bufs[s]], row_bufs[s], g_sems[s])
            if c >= 1:
                p = c - 1
                gathers[p].wait()
                writes[p] = pltpu.async_copy(
                    row_bufs[p % 2], out_hbm.at[pl.ds(base + p * chunk, chunk)],
                    o_sems[p % 2])
        last = nchunk - 1
        gathers[last].wait()
        writes[last] = pltpu.async_copy(
            row_bufs[last % 2], out_hbm.at[pl.ds(base + last * chunk, chunk)],
            o_sems[last % 2])
        for c in (last - 1, last):
            if c >= 0 and c not in (last - 2,):
                writes[c].wait()

    return gk(emb, idx)


def kernel(z, embeddings):
    z_flat = z.reshape(-1, _D)
    z2 = jnp.sum(z_flat ** 2, axis=1)
    emb_t = embeddings.T
    idx = _compute_indices(z_flat, emb_t, z2)
    q = _gather_rows(embeddings, idx)
    return q.reshape(z.shape)
